# baseline (speedup 1.0000x reference)
import functools

import jax
import jax.numpy as jnp
from jax import lax
from jax.experimental import pallas as pl
from jax.experimental.pallas import tpu as pltpu
from jax.experimental.pallas import tpu_sc as plsc

NUM_CORES = 2
NUM_SUBCORES = 16
NUM_TILES = NUM_CORES * NUM_SUBCORES
ROW_BLOCK = 1000


def _row_partition(n_nodes):
  rpt = (n_nodes // NUM_SUBCORES) & ~7
  tail_off = rpt * NUM_SUBCORES
  tail = n_nodes - tail_off
  return rpt, tail_off, tail


def _copy_row_slices(src, dst, s, n_nodes):
  rpt, tail_off, tail = _row_partition(n_nodes)
  off = pl.multiple_of(s * rpt, 8)
  pltpu.sync_copy(src.at[pl.ds(off, rpt)], dst.at[pl.ds(off, rpt)])
  if tail:
    @pl.when(s == 0)
    def _():
      pltpu.sync_copy(src.at[pl.ds(tail_off, tail)],
                      dst.at[pl.ds(tail_off, tail)])


STEP_CHUNK = 128
NBUF = 3


def _sc_step_kernel(n_nodes, n_edges, d):
  ept = n_edges // NUM_TILES
  n_chunks = ept // STEP_CHUNK
  tail = ept - n_chunks * STEP_CHUNK
  n_rounds = n_chunks // NBUF
  assert n_chunks % NBUF == 0 and n_rounds % 2 == 0
  assert tail <= STEP_CHUNK and tail % 8 == 0
  mesh = plsc.VectorSubcoreMesh(core_axis_name="c", subcore_axis_name="s")

  scratch = ([pltpu.VMEM((NBUF, 2, STEP_CHUNK), jnp.int32) for _ in range(2)]
             + [pltpu.VMEM((STEP_CHUNK, d), jnp.float32) for _ in range(NBUF)]
             + [pltpu.VMEM((2, max(tail, 8)), jnp.int32),
                pltpu.VMEM_SHARED((n_nodes, d), jnp.float32)]
             + [pltpu.SemaphoreType.DMA for _ in range(2 * NBUF + 2)])

  @functools.partial(
      pl.kernel,
      out_type=jax.ShapeDtypeStruct((NUM_CORES, n_nodes, d), jnp.float32),
      mesh=mesh,
      scratch_types=scratch,
  )
  def step_kernel(g_hbm, ei_hbm, eit_hbm, zeros_hbm, out_hbm, *sc):
    ib = sc[0:2]
    rows = sc[2:2 + NBUF]
    tidx = sc[2 + NBUF]
    acc_sh = sc[3 + NBUF]
    gsem = sc[4 + NBUF:4 + 2 * NBUF]
    ssem = sc[4 + 2 * NBUF:4 + 3 * NBUF]
    isem = sc[4 + 3 * NBUF:]
    c = lax.axis_index("c")
    s = lax.axis_index("s")
    w = c * NUM_SUBCORES + s

    def drain_rows(sem, b):
      pltpu.make_async_copy(g_hbm.at[pl.ds(0, STEP_CHUNK)], rows[b], sem).wait()

    def drain_idx(p):
      pltpu.make_async_copy(ei_hbm.at[w, 0], ib[p], isem[p]).wait()

    _copy_row_slices(zeros_hbm, acc_sh, s, n_nodes)
    plsc.subcore_barrier()

    pltpu.sync_copy(ei_hbm.at[w, 0], ib[0])
    pltpu.async_copy(ei_hbm.at[w, 1], ib[1], isem[1])
    for b in range(NBUF):
      pltpu.async_copy(g_hbm.at[ib[0].at[b, 0]], rows[b], gsem[b])

    def round_pair(i, carry):
      for po in range(2):
        r = i * 2 + po
        p, q = po, 1 - po

        @pl.when(r + 1 < n_rounds)
        def _():
          drain_idx(q)

        for b in range(NBUF):
          drain_rows(gsem[b], b)
          pltpu.async_copy(rows[b], acc_sh.at[ib[p].at[b, 1]], ssem[b],
                           add=True)
        for b in range(NBUF):
          drain_rows(ssem[b], b)

          @pl.when(r + 1 < n_rounds)
          def _():
            pltpu.async_copy(g_hbm.at[ib[q].at[b, 0]], rows[b], gsem[b])

        @pl.when(r + 2 < n_rounds)
        def _():
          pltpu.async_copy(ei_hbm.at[w, r + 2], ib[p], isem[p])

      return carry

    lax.fori_loop(0, n_rounds // 2, round_pair, 0)

    if tail:
      pltpu.sync_copy(eit_hbm.at[w], tidx)
      rows_t = rows[0].at[pl.ds(0, tail)]
      pltpu.async_copy(g_hbm.at[tidx.at[0, pl.ds(0, tail)]], rows_t,
                       gsem[0]).wait()
      pltpu.sync_copy(rows_t, acc_sh.at[tidx.at[1, pl.ds(0, tail)]], add=True)

    plsc.subcore_barrier()
    _copy_row_slices(acc_sh, out_hbm.at[c], s, n_nodes)

  return step_kernel


def _tc_init(x, w1, b1, w2, b2, deg_p, gamma0):
  n, d_in = x.shape
  d_out = w2.shape[1]
  grid = (n // ROW_BLOCK,)

  def body(x_ref, w1_ref, b1_ref, w2_ref, b2_ref, dp_ref, g0m_ref,
           out0_ref, g0_ref, dinv_ref):
    h = jnp.dot(x_ref[...], w1_ref[...], preferred_element_type=jnp.float32)
    h = jnp.maximum(h + b1_ref[...], 0.0)
    h = jnp.dot(h, w2_ref[...], preferred_element_type=jnp.float32)
    h = h + b2_ref[...]
    deg = dp_ref[0, :, 0:1] + dp_ref[1, :, 0:1]
    dinv = jnp.where(deg > 0.0, lax.rsqrt(deg), 0.0)
    out0_ref[...] = g0m_ref[0, 0] * h
    g0_ref[...] = dinv * h
    dinv_ref[...] = dinv

  return pl.pallas_call(
      body,
      grid=grid,
      in_specs=[
          pl.BlockSpec((ROW_BLOCK, d_in), lambda i: (i, 0)),
          pl.BlockSpec((d_in, w1.shape[1]), lambda i: (0, 0)),
          pl.BlockSpec((1, w1.shape[1]), lambda i: (0, 0)),
          pl.BlockSpec((w1.shape[1], d_out), lambda i: (0, 0)),
          pl.BlockSpec((1, d_out), lambda i: (0, 0)),
          pl.BlockSpec((NUM_CORES, ROW_BLOCK, d_out), lambda i: (0, i, 0)),
          pl.BlockSpec((1, 1), lambda i: (0, 0)),
      ],
      out_specs=[
          pl.BlockSpec((ROW_BLOCK, d_out), lambda i: (i, 0)),
          pl.BlockSpec((ROW_BLOCK, d_out), lambda i: (i, 0)),
          pl.BlockSpec((ROW_BLOCK, 1), lambda i: (i, 0)),
      ],
      out_shape=[
          jax.ShapeDtypeStruct((n, d_out), jnp.float32),
          jax.ShapeDtypeStruct((n, d_out), jnp.float32),
          jax.ShapeDtypeStruct((n, 1), jnp.float32),
      ],
  )(x, w1, b1, w2, b2, deg_p, gamma0)


def _tc_merge(s_p, dinv, out_prev, gamma_k):
  n, d = out_prev.shape
  grid = (n // ROW_BLOCK,)

  def body(sp_ref, dinv_ref, outp_ref, gk_ref, out_ref, g_ref):
    sblk = sp_ref[0] + sp_ref[1]
    dv = dinv_ref[...]
    h = dv * sblk
    out_ref[...] = outp_ref[...] + gk_ref[0, 0] * h
    g_ref[...] = dv * h

  return pl.pallas_call(
      body,
      grid=grid,
      in_specs=[
          pl.BlockSpec((NUM_CORES, ROW_BLOCK, d), lambda i: (0, i, 0)),
          pl.BlockSpec((ROW_BLOCK, 1), lambda i: (i, 0)),
          pl.BlockSpec((ROW_BLOCK, d), lambda i: (i, 0)),
          pl.BlockSpec((1, 1), lambda i: (0, 0)),
      ],
      out_specs=[
          pl.BlockSpec((ROW_BLOCK, d), lambda i: (i, 0)),
          pl.BlockSpec((ROW_BLOCK, d), lambda i: (i, 0)),
      ],
      out_shape=[
          jax.ShapeDtypeStruct((n, d), jnp.float32),
          jax.ShapeDtypeStruct((n, d), jnp.float32),
      ],
  )(s_p, dinv, out_prev, gamma_k)


def kernel(x, edge_index, W1, b1, W2, b2, gamma):
  n, _ = x.shape
  d = W2.shape[1]
  e = edge_index.shape[1]
  k_steps = gamma.shape[0] - 1
  row = edge_index[0]
  col = edge_index[1]

  zeros_nd = jnp.zeros((n, d), jnp.float32)
  ones_nd = jnp.ones((n, d), jnp.float32)

  ept = e // NUM_TILES
  n_chunks = ept // STEP_CHUNK
  main_e = n_chunks * STEP_CHUNK
  r2 = row.reshape(NUM_TILES, ept)
  c2 = col.reshape(NUM_TILES, ept)
  ei_main = jnp.stack([r2[:, :main_e].reshape(NUM_TILES, n_chunks, STEP_CHUNK),
                       c2[:, :main_e].reshape(NUM_TILES, n_chunks, STEP_CHUNK)],
                      axis=2)
  ei_main = ei_main.reshape(NUM_TILES, n_chunks // NBUF, NBUF, 2, STEP_CHUNK)
  tail = ept - main_e
  if tail:
    ei_tail = jnp.stack([r2[:, main_e:], c2[:, main_e:]], axis=1)
  else:
    ei_tail = jnp.zeros((NUM_TILES, 2, 8), jnp.int32)

  step = _sc_step_kernel(n, e, d)
  deg_p = step(ones_nd, ei_main, ei_tail, zeros_nd)
  out, g, dinv = _tc_init(x, W1, b1.reshape(1, -1), W2, b2.reshape(1, -1),
                          deg_p, gamma[0].reshape(1, 1))
  def body(k, carry):
    out_c, g_c = carry
    s_p = step(g_c, ei_main, ei_tail, zeros_nd)
    gk = lax.dynamic_slice(gamma, (k,), (1,)).reshape(1, 1)
    return _tc_merge(s_p, dinv, out_c, gk)

  out, g = lax.fori_loop(1, k_steps + 1, body, (out, g))
  return out


# speedup vs baseline: 1.0541x; 1.0541x over previous
import functools

import jax
import jax.numpy as jnp
from jax import lax
from jax.experimental import pallas as pl
from jax.experimental.pallas import tpu as pltpu
from jax.experimental.pallas import tpu_sc as plsc

NUM_CORES = 2
NUM_SUBCORES = 16
NUM_TILES = NUM_CORES * NUM_SUBCORES
ROW_BLOCK = 1000


def _row_partition(n_nodes):
  rpt = (n_nodes // NUM_SUBCORES) & ~7
  tail_off = rpt * NUM_SUBCORES
  tail = n_nodes - tail_off
  return rpt, tail_off, tail


def _copy_row_slices(src, dst, s, n_nodes):
  rpt, tail_off, tail = _row_partition(n_nodes)
  off = pl.multiple_of(s * rpt, 8)
  pltpu.sync_copy(src.at[pl.ds(off, rpt)], dst.at[pl.ds(off, rpt)])
  if tail:
    @pl.when(s == 0)
    def _():
      pltpu.sync_copy(src.at[pl.ds(tail_off, tail)],
                      dst.at[pl.ds(tail_off, tail)])


STEP_CHUNK = 96
NBUF = 4


def _sc_step_kernel(n_nodes, n_edges, d):
  ept = n_edges // NUM_TILES
  n_chunks = ept // STEP_CHUNK
  tail = ept - n_chunks * STEP_CHUNK
  n_rounds = n_chunks // NBUF
  assert n_chunks % NBUF == 0 and tail <= STEP_CHUNK and tail % 8 == 0
  mesh = plsc.VectorSubcoreMesh(core_axis_name="c", subcore_axis_name="s")

  scratch = ([pltpu.VMEM((2, STEP_CHUNK), jnp.int32) for _ in range(NBUF)]
             + [pltpu.VMEM((STEP_CHUNK, d), jnp.float32) for _ in range(NBUF)]
             + [pltpu.VMEM((2, max(tail, 8)), jnp.int32),
                pltpu.VMEM_SHARED((n_nodes, d), jnp.float32)]
             + [pltpu.SemaphoreType.DMA for _ in range(2 * NBUF)])

  @functools.partial(
      pl.kernel,
      out_type=jax.ShapeDtypeStruct((NUM_CORES, n_nodes, d), jnp.float32),
      mesh=mesh,
      scratch_types=scratch,
  )
  def step_kernel(g_hbm, ei_hbm, eit_hbm, zeros_hbm, out_hbm, *sc):
    idx = sc[0:NBUF]
    rows = sc[NBUF:2 * NBUF]
    tidx = sc[2 * NBUF]
    acc_sh = sc[2 * NBUF + 1]
    gsem = sc[2 * NBUF + 2:2 * NBUF + 2 + NBUF]
    ssem = sc[2 * NBUF + 2 + NBUF:]
    c = lax.axis_index("c")
    s = lax.axis_index("s")
    w = c * NUM_SUBCORES + s

    def drain(sem, b):
      pltpu.make_async_copy(g_hbm.at[pl.ds(0, STEP_CHUNK)], rows[b], sem).wait()

    _copy_row_slices(zeros_hbm, acc_sh, s, n_nodes)
    plsc.subcore_barrier()

    for b in range(NBUF):
      pltpu.sync_copy(ei_hbm.at[w, b], idx[b])
      pltpu.async_copy(g_hbm.at[idx[b].at[0]], rows[b], gsem[b])

    def round_(i, carry):
      for b in range(NBUF):
        drain(gsem[b], b)
        pltpu.async_copy(rows[b], acc_sh.at[idx[b].at[1]], ssem[b], add=True)

      @pl.when(i < n_rounds - 1)
      def _():
        for b in range(NBUF):
          drain(ssem[b], b)
          pltpu.sync_copy(ei_hbm.at[w, i * NBUF + NBUF + b], idx[b])
          pltpu.async_copy(g_hbm.at[idx[b].at[0]], rows[b], gsem[b])

      return carry

    lax.fori_loop(0, n_rounds, round_, 0)
    for b in range(NBUF):
      drain(ssem[b], b)

    if tail:
      pltpu.sync_copy(eit_hbm.at[w], tidx)
      rows_t = rows[0].at[pl.ds(0, tail)]
      pltpu.async_copy(g_hbm.at[tidx.at[0, pl.ds(0, tail)]], rows_t,
                       gsem[0]).wait()
      pltpu.sync_copy(rows_t, acc_sh.at[tidx.at[1, pl.ds(0, tail)]], add=True)

    plsc.subcore_barrier()
    _copy_row_slices(acc_sh, out_hbm.at[c], s, n_nodes)

  return step_kernel


def _tc_init(x, w1, b1, w2, b2, deg_p, gamma0):
  n, d_in = x.shape
  d_out = w2.shape[1]
  grid = (n // ROW_BLOCK,)

  def body(x_ref, w1_ref, b1_ref, w2_ref, b2_ref, dp_ref, g0m_ref,
           out0_ref, g0_ref, dinv_ref):
    h = jnp.dot(x_ref[...], w1_ref[...], preferred_element_type=jnp.float32)
    h = jnp.maximum(h + b1_ref[...], 0.0)
    h = jnp.dot(h, w2_ref[...], preferred_element_type=jnp.float32)
    h = h + b2_ref[...]
    deg = dp_ref[0, :, 0:1] + dp_ref[1, :, 0:1]
    dinv = jnp.where(deg > 0.0, lax.rsqrt(deg), 0.0)
    out0_ref[...] = g0m_ref[0, 0] * h
    g0_ref[...] = dinv * h
    dinv_ref[...] = dinv

  return pl.pallas_call(
      body,
      grid=grid,
      in_specs=[
          pl.BlockSpec((ROW_BLOCK, d_in), lambda i: (i, 0)),
          pl.BlockSpec((d_in, w1.shape[1]), lambda i: (0, 0)),
          pl.BlockSpec((1, w1.shape[1]), lambda i: (0, 0)),
          pl.BlockSpec((w1.shape[1], d_out), lambda i: (0, 0)),
          pl.BlockSpec((1, d_out), lambda i: (0, 0)),
          pl.BlockSpec((NUM_CORES, ROW_BLOCK, d_out), lambda i: (0, i, 0)),
          pl.BlockSpec((1, 1), lambda i: (0, 0)),
      ],
      out_specs=[
          pl.BlockSpec((ROW_BLOCK, d_out), lambda i: (i, 0)),
          pl.BlockSpec((ROW_BLOCK, d_out), lambda i: (i, 0)),
          pl.BlockSpec((ROW_BLOCK, 1), lambda i: (i, 0)),
      ],
      out_shape=[
          jax.ShapeDtypeStruct((n, d_out), jnp.float32),
          jax.ShapeDtypeStruct((n, d_out), jnp.float32),
          jax.ShapeDtypeStruct((n, 1), jnp.float32),
      ],
  )(x, w1, b1, w2, b2, deg_p, gamma0)


def _tc_merge(s_p, dinv, out_prev, gamma_k):
  n, d = out_prev.shape
  grid = (n // ROW_BLOCK,)

  def body(sp_ref, dinv_ref, outp_ref, gk_ref, out_ref, g_ref):
    sblk = sp_ref[0] + sp_ref[1]
    dv = dinv_ref[...]
    h = dv * sblk
    out_ref[...] = outp_ref[...] + gk_ref[0, 0] * h
    g_ref[...] = dv * h

  return pl.pallas_call(
      body,
      grid=grid,
      in_specs=[
          pl.BlockSpec((NUM_CORES, ROW_BLOCK, d), lambda i: (0, i, 0)),
          pl.BlockSpec((ROW_BLOCK, 1), lambda i: (i, 0)),
          pl.BlockSpec((ROW_BLOCK, d), lambda i: (i, 0)),
          pl.BlockSpec((1, 1), lambda i: (0, 0)),
      ],
      out_specs=[
          pl.BlockSpec((ROW_BLOCK, d), lambda i: (i, 0)),
          pl.BlockSpec((ROW_BLOCK, d), lambda i: (i, 0)),
      ],
      out_shape=[
          jax.ShapeDtypeStruct((n, d), jnp.float32),
          jax.ShapeDtypeStruct((n, d), jnp.float32),
      ],
  )(s_p, dinv, out_prev, gamma_k)


def kernel(x, edge_index, W1, b1, W2, b2, gamma):
  n, _ = x.shape
  d = W2.shape[1]
  e = edge_index.shape[1]
  k_steps = gamma.shape[0] - 1
  row = edge_index[0]
  col = edge_index[1]

  zeros_nd = jnp.zeros((n, d), jnp.float32)
  ones_nd = jnp.ones((n, d), jnp.float32)

  ept = e // NUM_TILES
  n_chunks = ept // STEP_CHUNK
  main_e = n_chunks * STEP_CHUNK
  r2 = row.reshape(NUM_TILES, ept)
  c2 = col.reshape(NUM_TILES, ept)
  ei_main = jnp.stack([r2[:, :main_e].reshape(NUM_TILES, n_chunks, STEP_CHUNK),
                       c2[:, :main_e].reshape(NUM_TILES, n_chunks, STEP_CHUNK)],
                      axis=2)
  tail = ept - main_e
  if tail:
    ei_tail = jnp.stack([r2[:, main_e:], c2[:, main_e:]], axis=1)
  else:
    ei_tail = jnp.zeros((NUM_TILES, 2, 8), jnp.int32)

  step = _sc_step_kernel(n, e, d)
  deg_p = step(ones_nd, ei_main, ei_tail, zeros_nd)
  out, g, dinv = _tc_init(x, W1, b1.reshape(1, -1), W2, b2.reshape(1, -1),
                          deg_p, gamma[0].reshape(1, 1))
  def body(k, carry):
    out_c, g_c = carry
    s_p = step(g_c, ei_main, ei_tail, zeros_nd)
    gk = lax.dynamic_slice(gamma, (k,), (1,)).reshape(1, 1)
    return _tc_merge(s_p, dinv, out_c, gk)

  out, g = lax.fori_loop(1, k_steps + 1, body, (out, g))
  return out
